# Initial kernel scaffold; baseline (speedup 1.0000x reference)
import jax
import jax.numpy as jnp
from jax.experimental import pallas as pl


def kernel(x, group, W1, b1, W2, b2):
    raise NotImplementedError("write your pallas kernel here")


# speedup vs baseline: 4.7406x; 4.7406x over previous
import functools

import jax
import jax.numpy as jnp
from jax import lax
from jax.experimental import pallas as pl
from jax.experimental.pallas import tpu as pltpu
from jax.experimental.pallas import tpu_sc as plsc

N, D, G, C = 320000, 128, 10000, 64
NC, NS = 2, 16
NW = NC * NS
CHUNK = 64
SUP = 8
NSUPER = N // (CHUNK * SUP)
CW = 16
NBLK = (G + CHUNK - 1) // CHUNK
BPS = (NBLK + NS - 1) // NS


def _sc_segment_sums(x, grp2d):
    mesh = plsc.VectorSubcoreMesh(
        core_axis_name="c", subcore_axis_name="s", num_cores=NC, num_subcores=NS
    )

    @functools.partial(
        pl.kernel,
        out_type=(
            jax.ShapeDtypeStruct((NC, G, D), jnp.float32),
            jax.ShapeDtypeStruct((NC, G, CW), jnp.float32),
        ),
        mesh=mesh,
        compiler_params=pltpu.CompilerParams(use_tc_tiling_on_sc=False),
        scratch_types=[
            pltpu.VMEM((CHUNK, D), jnp.float32),
            pltpu.VMEM((SUP, CHUNK), jnp.int32),
            pltpu.VMEM((CHUNK, CW), jnp.float32),
            pltpu.VMEM((CHUNK, CW), jnp.float32),
            pltpu.VMEM_SHARED((G, D), jnp.float32),
            pltpu.VMEM_SHARED((G, CW), jnp.float32),
        ],
    )
    def k(x_hbm, grp_hbm, osum_hbm, ocnt_hbm, data_v, idx_v, ones_v, cb_v,
          ssum, scnt):
        cid = lax.axis_index("c")
        sid = lax.axis_index("s")
        wid = sid * NC + cid
        lanes = lax.iota(jnp.int32, 16)

        def fill(r, _):
            for kk in range(D // 16):
                data_v[r, pl.ds(kk * 16, 16)] = jnp.zeros((16,), jnp.float32)
            ones_v[r, :] = jnp.ones((CW,), jnp.float32)
            cb_v[r, :] = jnp.zeros((CW,), jnp.float32)
            return 0
        lax.fori_loop(0, CHUNK, fill, 0)

        def fill_idx_row(start):
            for sl in range(CHUNK // 16):
                idx_v[0, pl.ds(sl * 16, 16)] = start + sl * 16 + lanes

        def blk_start(b):
            return jnp.minimum(b * CHUNK, G - CHUNK).astype(jnp.int32)

        def init_body(t, _):
            b = sid + t * NS

            @pl.when(b < NBLK)
            def _():
                fill_idx_row(blk_start(b))
                pltpu.sync_copy(data_v, ssum.at[idx_v.at[0]])
                pltpu.sync_copy(cb_v, scnt.at[idx_v.at[0]])
            return 0

        lax.fori_loop(0, BPS, init_body, 0)
        plsc.subcore_barrier()

        nloop = (NSUPER + NW - 1) // NW

        def body(j, _):
            s = wid + j * NW

            @pl.when(s < NSUPER)
            def _():
                pltpu.sync_copy(grp_hbm.at[pl.ds(s * SUP, SUP)], idx_v)
                for kk in range(SUP):
                    base = s * (SUP * CHUNK) + kk * CHUNK
                    pltpu.sync_copy(x_hbm.at[pl.ds(base, CHUNK)], data_v)
                    pltpu.sync_copy(data_v, ssum.at[idx_v.at[kk]], add=True)
                    pltpu.sync_copy(ones_v, scnt.at[idx_v.at[kk]], add=True)
            return 0

        lax.fori_loop(0, nloop, body, 0)
        plsc.subcore_barrier()

        def drain_body(t, _):
            b = sid + t * NS

            @pl.when(b < NBLK)
            def _():
                start = blk_start(b)
                fill_idx_row(start)
                pltpu.sync_copy(ssum.at[idx_v.at[0]], data_v)
                pltpu.sync_copy(data_v, osum_hbm.at[cid, pl.ds(start, CHUNK)])
                pltpu.sync_copy(scnt.at[idx_v.at[0]], cb_v)
                pltpu.sync_copy(cb_v, ocnt_hbm.at[cid, pl.ds(start, CHUNK)])
            return 0

        lax.fori_loop(0, BPS, drain_body, 0)

    return k(x, grp2d)


def _mlp_body(psum_ref, pcnt_ref, w1_ref, b1_ref, w2_ref, b2_ref, out_ref):
    s = psum_ref[0] + psum_ref[1]
    cnt = pcnt_ref[0, :, 0:1] + pcnt_ref[1, :, 0:1]
    means = s / jnp.maximum(cnt, 1.0)
    h = jnp.dot(means, w1_ref[...], preferred_element_type=jnp.float32)
    h = h + b1_ref[...][None, :]
    h = 0.5 * h * (1.0 + lax.erf(h * 0.7071067811865476))
    o = jnp.dot(h, w2_ref[...], preferred_element_type=jnp.float32)
    out_ref[...] = o + b2_ref[...][None, :]


def _tc_mlp(psum, pcnt, W1, b1, W2, b2):
    BG = 1000
    return pl.pallas_call(
        _mlp_body,
        grid=(G // BG,),
        in_specs=[
            pl.BlockSpec((NC, BG, D), lambda i: (0, i, 0)),
            pl.BlockSpec((NC, BG, CW), lambda i: (0, i, 0)),
            pl.BlockSpec((D, D), lambda i: (0, 0)),
            pl.BlockSpec((D,), lambda i: (0,)),
            pl.BlockSpec((D, C), lambda i: (0, 0)),
            pl.BlockSpec((C,), lambda i: (0,)),
        ],
        out_specs=pl.BlockSpec((BG, C), lambda i: (i, 0)),
        out_shape=jax.ShapeDtypeStruct((G, C), jnp.float32),
    )(psum, pcnt, W1, b1, W2, b2)


def kernel(x, group, W1, b1, W2, b2):
    grp2d = group.reshape(N // CHUNK, CHUNK)
    psum, pcnt = _sc_segment_sums(x, grp2d)
    return _tc_mlp(psum, pcnt, W1, b1, W2, b2)


# speedup vs baseline: 7.0208x; 1.4810x over previous
import functools

import jax
import jax.numpy as jnp
from jax import lax
from jax.experimental import pallas as pl
from jax.experimental.pallas import tpu as pltpu
from jax.experimental.pallas import tpu_sc as plsc

N, D, G, C = 320000, 128, 10000, 64
NC, NS = 2, 16
NW = NC * NS
CHUNK = 64
SUP = 8
NSUPER = N // (CHUNK * SUP)
CW = 16
NBLK = (G + CHUNK - 1) // CHUNK
BPS = (NBLK + NS - 1) // NS


def _sc_segment_sums(x, grp2d):
    mesh = plsc.VectorSubcoreMesh(
        core_axis_name="c", subcore_axis_name="s", num_cores=NC, num_subcores=NS
    )

    @functools.partial(
        pl.kernel,
        out_type=(
            jax.ShapeDtypeStruct((NC, G, D), jnp.float32),
            jax.ShapeDtypeStruct((NC, G, CW), jnp.float32),
        ),
        mesh=mesh,
        compiler_params=pltpu.CompilerParams(use_tc_tiling_on_sc=False),
        scratch_types=[
            pltpu.VMEM((CHUNK, D), jnp.float32),
            pltpu.VMEM((CHUNK, D), jnp.float32),
            pltpu.VMEM((SUP, CHUNK), jnp.int32),
            pltpu.VMEM((CHUNK, CW), jnp.float32),
            pltpu.VMEM((CHUNK, CW), jnp.float32),
            pltpu.VMEM_SHARED((G, D), jnp.float32),
            pltpu.VMEM_SHARED((G, CW), jnp.float32),
            pltpu.SemaphoreType.DMA,
            pltpu.SemaphoreType.DMA,
            pltpu.SemaphoreType.DMA,
        ],
    )
    def k(x_hbm, grp_hbm, osum_hbm, ocnt_hbm, data_v, data2_v, idx_v, ones_v,
          cb_v, ssum, scnt, semx0, semx1, semc):
        cid = lax.axis_index("c")
        sid = lax.axis_index("s")
        wid = sid * NC + cid
        lanes = lax.iota(jnp.int32, 16)

        def fill(r, _):
            for kk in range(D // 16):
                data_v[r, pl.ds(kk * 16, 16)] = jnp.zeros((16,), jnp.float32)
            ones_v[r, :] = jnp.ones((CW,), jnp.float32)
            cb_v[r, :] = jnp.zeros((CW,), jnp.float32)
            return 0
        lax.fori_loop(0, CHUNK, fill, 0)

        def fill_idx_row(start):
            for sl in range(CHUNK // 16):
                idx_v[0, pl.ds(sl * 16, 16)] = start + sl * 16 + lanes

        def blk_start(b):
            return jnp.minimum(b * CHUNK, G - CHUNK).astype(jnp.int32)

        def init_body(t, _):
            b = sid + t * NS

            @pl.when(b < NBLK)
            def _():
                fill_idx_row(blk_start(b))
                pltpu.sync_copy(data_v, ssum.at[idx_v.at[0]])
                pltpu.sync_copy(cb_v, scnt.at[idx_v.at[0]])
            return 0

        lax.fori_loop(0, BPS, init_body, 0)
        plsc.subcore_barrier()

        nloop = (NSUPER + NW - 1) // NW

        bufs = (data_v, data2_v)
        sems = (semx0, semx1)

        def body(j, _):
            s = wid + j * NW

            @pl.when(s < NSUPER)
            def _():
                pltpu.sync_copy(grp_hbm.at[pl.ds(s * SUP, SUP)], idx_v)
                base = s * (SUP * CHUNK)
                pltpu.async_copy(x_hbm.at[pl.ds(base, CHUNK)], bufs[0], sems[0])
                for kk in range(SUP):
                    b = kk % 2
                    if kk + 1 < SUP:
                        nbase = base + (kk + 1) * CHUNK
                        pltpu.async_copy(x_hbm.at[pl.ds(nbase, CHUNK)],
                                         bufs[1 - b], sems[1 - b])
                    pltpu.make_async_copy(
                        x_hbm.at[pl.ds(base + kk * CHUNK, CHUNK)],
                        bufs[b], sems[b]).wait()
                    pltpu.sync_copy(bufs[b], ssum.at[idx_v.at[kk]], add=True)
                    pltpu.async_copy(ones_v, scnt.at[idx_v.at[kk]], semc,
                                     add=True)
                for kk in range(SUP):
                    pltpu.make_async_copy(ones_v, scnt.at[idx_v.at[kk]],
                                          semc).wait()
            return 0

        lax.fori_loop(0, nloop, body, 0)
        plsc.subcore_barrier()

        def drain_body(t, _):
            b = sid + t * NS

            @pl.when(b < NBLK)
            def _():
                start = blk_start(b)
                fill_idx_row(start)
                pltpu.sync_copy(ssum.at[idx_v.at[0]], data_v)
                pltpu.sync_copy(data_v, osum_hbm.at[cid, pl.ds(start, CHUNK)])
                pltpu.sync_copy(scnt.at[idx_v.at[0]], cb_v)
                pltpu.sync_copy(cb_v, ocnt_hbm.at[cid, pl.ds(start, CHUNK)])
            return 0

        lax.fori_loop(0, BPS, drain_body, 0)

    return k(x, grp2d)


def _mlp_body(psum_ref, pcnt_ref, w1_ref, b1_ref, w2_ref, b2_ref, out_ref):
    s = psum_ref[0] + psum_ref[1]
    cnt = pcnt_ref[0, :, 0:1] + pcnt_ref[1, :, 0:1]
    means = s / jnp.maximum(cnt, 1.0)
    h = jnp.dot(means, w1_ref[...], preferred_element_type=jnp.float32)
    h = h + b1_ref[...][None, :]
    h = 0.5 * h * (1.0 + lax.erf(h * 0.7071067811865476))
    o = jnp.dot(h, w2_ref[...], preferred_element_type=jnp.float32)
    out_ref[...] = o + b2_ref[...][None, :]


def _tc_mlp(psum, pcnt, W1, b1, W2, b2):
    BG = 1000
    return pl.pallas_call(
        _mlp_body,
        grid=(G // BG,),
        in_specs=[
            pl.BlockSpec((NC, BG, D), lambda i: (0, i, 0)),
            pl.BlockSpec((NC, BG, CW), lambda i: (0, i, 0)),
            pl.BlockSpec((D, D), lambda i: (0, 0)),
            pl.BlockSpec((D,), lambda i: (0,)),
            pl.BlockSpec((D, C), lambda i: (0, 0)),
            pl.BlockSpec((C,), lambda i: (0,)),
        ],
        out_specs=pl.BlockSpec((BG, C), lambda i: (i, 0)),
        out_shape=jax.ShapeDtypeStruct((G, C), jnp.float32),
    )(psum, pcnt, W1, b1, W2, b2)


def kernel(x, group, W1, b1, W2, b2):
    grp2d = group.reshape(N // CHUNK, CHUNK)
    psum, pcnt = _sc_segment_sums(x, grp2d)
    return _tc_mlp(psum, pcnt, W1, b1, W2, b2)


# speedup vs baseline: 7.1982x; 1.0253x over previous
import functools

import jax
import jax.numpy as jnp
from jax import lax
from jax.experimental import pallas as pl
from jax.experimental.pallas import tpu as pltpu
from jax.experimental.pallas import tpu_sc as plsc

N, D, G, C = 320000, 128, 10000, 64
NC, NS = 2, 16
NW = NC * NS
CHUNK = 128
SUP = 4
NSUPER = N // (CHUNK * SUP)
CW = 16
NBLK = (G + CHUNK - 1) // CHUNK
BPS = (NBLK + NS - 1) // NS


def _sc_segment_sums(x, grp2d):
    mesh = plsc.VectorSubcoreMesh(
        core_axis_name="c", subcore_axis_name="s", num_cores=NC, num_subcores=NS
    )

    @functools.partial(
        pl.kernel,
        out_type=(
            jax.ShapeDtypeStruct((NC, G, D), jnp.float32),
            jax.ShapeDtypeStruct((NC, G, CW), jnp.float32),
        ),
        mesh=mesh,
        compiler_params=pltpu.CompilerParams(use_tc_tiling_on_sc=False),
        scratch_types=[
            pltpu.VMEM((CHUNK, D), jnp.float32),
            pltpu.VMEM((CHUNK, D), jnp.float32),
            pltpu.VMEM((SUP, CHUNK), jnp.int32),
            pltpu.VMEM((CHUNK, CW), jnp.float32),
            pltpu.VMEM((CHUNK, CW), jnp.float32),
            pltpu.VMEM_SHARED((G, D), jnp.float32),
            pltpu.VMEM_SHARED((G, CW), jnp.float32),
            pltpu.SemaphoreType.DMA,
            pltpu.SemaphoreType.DMA,
            pltpu.SemaphoreType.DMA,
            pltpu.SemaphoreType.DMA,
        ],
    )
    def k(x_hbm, grp_hbm, osum_hbm, ocnt_hbm, data_v, data2_v, idx_v, ones_v,
          cb_v, ssum, scnt, semx0, semx1, sems, semc):
        cid = lax.axis_index("c")
        sid = lax.axis_index("s")
        wid = sid * NC + cid
        lanes = lax.iota(jnp.int32, 16)
        bufs = (data_v, data2_v)
        sems_x = (semx0, semx1)

        def fill(r, _):
            for kk in range(D // 16):
                data_v[r, pl.ds(kk * 16, 16)] = jnp.zeros((16,), jnp.float32)
            ones_v[r, :] = jnp.ones((CW,), jnp.float32)
            cb_v[r, :] = jnp.zeros((CW,), jnp.float32)
            return 0
        lax.fori_loop(0, CHUNK, fill, 0)

        def fill_idx_row(start):
            for sl in range(CHUNK // 16):
                idx_v[0, pl.ds(sl * 16, 16)] = start + sl * 16 + lanes

        def blk_start(b):
            return jnp.minimum(b * CHUNK, G - CHUNK).astype(jnp.int32)

        def init_body(t, _):
            b = sid + t * NS

            @pl.when(b < NBLK)
            def _():
                fill_idx_row(blk_start(b))
                pltpu.sync_copy(data_v, ssum.at[idx_v.at[0]])
                pltpu.sync_copy(cb_v, scnt.at[idx_v.at[0]])
            return 0

        lax.fori_loop(0, BPS, init_body, 0)
        plsc.subcore_barrier()

        nloop = (NSUPER + NW - 1) // NW

        def body(j, _):
            s = wid + j * NW

            @pl.when(s < NSUPER)
            def _():
                pltpu.sync_copy(grp_hbm.at[pl.ds(s * SUP, SUP)], idx_v)
                base = s * (SUP * CHUNK)
                pltpu.async_copy(x_hbm.at[pl.ds(base, CHUNK)], bufs[0],
                                 sems_x[0])
                for kk in range(SUP):
                    b = kk % 2
                    if kk >= 1:
                        pltpu.make_async_copy(bufs[1 - b],
                                              ssum.at[idx_v.at[kk - 1]],
                                              sems).wait()
                    if kk + 1 < SUP:
                        nbase = base + (kk + 1) * CHUNK
                        pltpu.async_copy(x_hbm.at[pl.ds(nbase, CHUNK)],
                                         bufs[1 - b], sems_x[1 - b])
                    pltpu.make_async_copy(
                        x_hbm.at[pl.ds(base + kk * CHUNK, CHUNK)],
                        bufs[b], sems_x[b]).wait()
                    pltpu.async_copy(bufs[b], ssum.at[idx_v.at[kk]], sems,
                                     add=True)
                    pltpu.async_copy(ones_v, scnt.at[idx_v.at[kk]], semc,
                                     add=True)
                pltpu.make_async_copy(bufs[(SUP - 1) % 2],
                                      ssum.at[idx_v.at[SUP - 1]], sems).wait()
                for kk in range(SUP):
                    pltpu.make_async_copy(ones_v, scnt.at[idx_v.at[kk]],
                                          semc).wait()
            return 0

        lax.fori_loop(0, nloop, body, 0)
        plsc.subcore_barrier()

        def drain_body(t, _):
            b = sid + t * NS

            @pl.when(b < NBLK)
            def _():
                start = blk_start(b)
                fill_idx_row(start)
                pltpu.sync_copy(ssum.at[idx_v.at[0]], data_v)
                pltpu.sync_copy(data_v, osum_hbm.at[cid, pl.ds(start, CHUNK)])
                pltpu.sync_copy(scnt.at[idx_v.at[0]], cb_v)
                pltpu.sync_copy(cb_v, ocnt_hbm.at[cid, pl.ds(start, CHUNK)])
            return 0

        lax.fori_loop(0, BPS, drain_body, 0)

    return k(x, grp2d)


def _mlp_body(psum_ref, pcnt_ref, w1_ref, b1_ref, w2_ref, b2_ref, out_ref):
    s = psum_ref[0] + psum_ref[1]
    cnt = pcnt_ref[0, :, 0:1] + pcnt_ref[1, :, 0:1]
    means = s / jnp.maximum(cnt, 1.0)
    h = jnp.dot(means, w1_ref[...], preferred_element_type=jnp.float32)
    h = h + b1_ref[...][None, :]
    h = 0.5 * h * (1.0 + lax.erf(h * 0.7071067811865476))
    o = jnp.dot(h, w2_ref[...], preferred_element_type=jnp.float32)
    out_ref[...] = o + b2_ref[...][None, :]


def _tc_mlp(psum, pcnt, W1, b1, W2, b2):
    BG = 1000
    return pl.pallas_call(
        _mlp_body,
        grid=(G // BG,),
        in_specs=[
            pl.BlockSpec((NC, BG, D), lambda i: (0, i, 0)),
            pl.BlockSpec((NC, BG, CW), lambda i: (0, i, 0)),
            pl.BlockSpec((D, D), lambda i: (0, 0)),
            pl.BlockSpec((D,), lambda i: (0,)),
            pl.BlockSpec((D, C), lambda i: (0, 0)),
            pl.BlockSpec((C,), lambda i: (0,)),
        ],
        out_specs=pl.BlockSpec((BG, C), lambda i: (i, 0)),
        out_shape=jax.ShapeDtypeStruct((G, C), jnp.float32),
    )(psum, pcnt, W1, b1, W2, b2)


def kernel(x, group, W1, b1, W2, b2):
    grp2d = group.reshape(N // CHUNK, CHUNK)
    psum, pcnt = _sc_segment_sums(x, grp2d)
    return _tc_mlp(psum, pcnt, W1, b1, W2, b2)


# speedup vs baseline: 8.7521x; 1.2159x over previous
import functools

import jax
import jax.numpy as jnp
from jax import lax
from jax.experimental import pallas as pl
from jax.experimental.pallas import tpu as pltpu
from jax.experimental.pallas import tpu_sc as plsc

N, D, G, C = 320000, 128, 10000, 64
NC, NS = 2, 16
NW = NC * NS
CHUNK = 128
SUP = 4
SROWS = SUP * CHUNK
NSUPER = N // SROWS
CW = 16
NBLK = (G + CHUNK - 1) // CHUNK
BPS = (NBLK + NS - 1) // NS
IDXR = max(SUP, BPS)


def _sc_segment_sums(x, grp2d):
    mesh = plsc.VectorSubcoreMesh(
        core_axis_name="c", subcore_axis_name="s", num_cores=NC, num_subcores=NS
    )

    @functools.partial(
        pl.kernel,
        out_type=(
            jax.ShapeDtypeStruct((NC, G, D), jnp.float32),
            jax.ShapeDtypeStruct((NC, G, CW), jnp.float32),
        ),
        mesh=mesh,
        compiler_params=pltpu.CompilerParams(use_tc_tiling_on_sc=False),
        scratch_types=[
            pltpu.VMEM((CHUNK, D), jnp.float32),
            pltpu.VMEM((CHUNK, D), jnp.float32),
            pltpu.VMEM((IDXR, CHUNK), jnp.int32),
            pltpu.VMEM((IDXR, CHUNK), jnp.int32),
            pltpu.VMEM((CHUNK, CW), jnp.float32),
            pltpu.VMEM((CHUNK, CW), jnp.float32),
            pltpu.VMEM_SHARED((G, D), jnp.float32),
            pltpu.VMEM_SHARED((G, CW), jnp.float32),
            pltpu.SemaphoreType.DMA,
            pltpu.SemaphoreType.DMA,
            pltpu.SemaphoreType.DMA,
            pltpu.SemaphoreType.DMA,
            pltpu.SemaphoreType.DMA,
            pltpu.SemaphoreType.DMA,
        ],
    )
    def k(x_hbm, grp_hbm, osum_hbm, ocnt_hbm, data_v, data2_v, idxa_v, idxb_v,
          ones_v, cb_v, ssum, scnt, semx0, semx1, sems, semc, semia, semib):
        cid = lax.axis_index("c")
        sid = lax.axis_index("s")
        wid = sid * NC + cid
        lanes = lax.iota(jnp.int32, 16)
        bufs = (data_v, data2_v)
        sems_x = (semx0, semx1)

        def fill(r, _):
            for kk in range(D // 16):
                data_v[r, pl.ds(kk * 16, 16)] = jnp.zeros((16,), jnp.float32)
            ones_v[r, :] = jnp.ones((CW,), jnp.float32)
            cb_v[r, :] = jnp.zeros((CW,), jnp.float32)
            return 0
        lax.fori_loop(0, CHUNK, fill, 0)

        def fill_idx_row(row, start):
            for sl in range(CHUNK // 16):
                idxa_v[row, pl.ds(sl * 16, 16)] = start + sl * 16 + lanes

        def blk_start(b):
            return jnp.minimum(b * CHUNK, G - CHUNK).astype(jnp.int32)

        for t in range(BPS):
            b = sid + t * NS

            @pl.when(b < NBLK)
            def _(t=t, b=b):
                fill_idx_row(t, blk_start(b))
                pltpu.async_copy(data_v, ssum.at[idxa_v.at[t]], sems)
                pltpu.async_copy(cb_v, scnt.at[idxa_v.at[t]], semc)

        for t in range(BPS):
            b = sid + t * NS

            @pl.when(b < NBLK)
            def _(t=t):
                pltpu.make_async_copy(data_v, ssum.at[idxa_v.at[t]], sems).wait()
                pltpu.make_async_copy(cb_v, scnt.at[idxa_v.at[t]], semc).wait()

        plsc.subcore_barrier()

        nloop = (NSUPER + NW - 1) // NW
        npair = (nloop + 1) // 2

        def idx_src(s):
            return grp_hbm.at[pl.ds(s * SUP, SUP), :]

        def x_src(s, kk):
            return x_hbm.at[pl.ds(s * SROWS + kk * CHUNK, CHUNK)]

        pltpu.async_copy(idx_src(wid), idxa_v.at[pl.ds(0, SUP)], semia)
        pltpu.async_copy(x_src(wid, 0), bufs[0], sems_x[0])

        def run_super(s, my_idx, my_sem, pf_s, pf_idx, pf_sem):
            @pl.when(s < NSUPER)
            def _():
                pltpu.make_async_copy(idx_src(s), my_idx.at[pl.ds(0, SUP)],
                                      my_sem).wait()

                @pl.when(pf_s < NSUPER)
                def _():
                    pltpu.async_copy(idx_src(pf_s), pf_idx.at[pl.ds(0, SUP)],
                                     pf_sem)
                for kk in range(SUP):
                    b = kk % 2
                    if kk >= 1:
                        pltpu.make_async_copy(bufs[1 - b],
                                              ssum.at[my_idx.at[kk - 1]],
                                              sems).wait()
                    if kk + 1 < SUP:
                        pltpu.async_copy(x_src(s, kk + 1), bufs[1 - b],
                                         sems_x[1 - b])
                    else:
                        @pl.when(s + NW < NSUPER)
                        def _():
                            pltpu.async_copy(x_src(s + NW, 0), bufs[1 - b],
                                             sems_x[1 - b])
                    pltpu.make_async_copy(x_src(s, kk), bufs[b],
                                          sems_x[b]).wait()
                    pltpu.async_copy(bufs[b], ssum.at[my_idx.at[kk]], sems,
                                     add=True)
                    pltpu.async_copy(ones_v, scnt.at[my_idx.at[kk]], semc,
                                     add=True)
                pltpu.make_async_copy(bufs[(SUP - 1) % 2],
                                      ssum.at[my_idx.at[SUP - 1]], sems).wait()
                for kk in range(SUP):
                    pltpu.make_async_copy(ones_v, scnt.at[my_idx.at[kk]],
                                          semc).wait()

        def body(jp, _):
            j0 = 2 * jp
            s0 = wid + j0 * NW
            s1 = wid + (j0 + 1) * NW
            s2 = wid + (j0 + 2) * NW
            run_super(s0, idxa_v, semia, s1, idxb_v, semib)
            run_super(s1, idxb_v, semib, s2, idxa_v, semia)
            return 0

        lax.fori_loop(0, npair, body, 0)
        plsc.subcore_barrier()

        cbufs = (cb_v, ones_v)
        sems_c = (semia, semib)

        for t in range(BPS):
            b = sid + t * NS

            @pl.when(b < NBLK)
            def _(t=t, b=b):
                fill_idx_row(t, blk_start(b))

        for t in range(BPS):
            b = sid + t * NS
            p = t % 2

            @pl.when(b < NBLK)
            def _(t=t, b=b, p=p):
                start = blk_start(b)
                if t >= 2:
                    pb = blk_start(sid + (t - 2) * NS)
                    pltpu.make_async_copy(
                        bufs[p], osum_hbm.at[cid, pl.ds(pb, CHUNK)],
                        sems_x[p]).wait()
                    pltpu.make_async_copy(
                        cbufs[p], ocnt_hbm.at[cid, pl.ds(pb, CHUNK)],
                        sems_c[p]).wait()
                pltpu.async_copy(ssum.at[idxa_v.at[t]], bufs[p], sems_x[p])
                pltpu.async_copy(scnt.at[idxa_v.at[t]], cbufs[p], sems_c[p])
                pltpu.make_async_copy(ssum.at[idxa_v.at[t]], bufs[p],
                                      sems_x[p]).wait()
                pltpu.async_copy(bufs[p], osum_hbm.at[cid, pl.ds(start, CHUNK)],
                                 sems_x[p])
                pltpu.make_async_copy(scnt.at[idxa_v.at[t]], cbufs[p],
                                      sems_c[p]).wait()
                pltpu.async_copy(cbufs[p],
                                 ocnt_hbm.at[cid, pl.ds(start, CHUNK)],
                                 sems_c[p])

        for t in range(BPS):
            b = sid + t * NS
            p = t % 2
            pending = jnp.logical_and(b < NBLK,
                                      (t + 2 >= BPS) | (sid + (t + 2) * NS >= NBLK))

            @pl.when(pending)
            def _(t=t, b=b, p=p):
                pb = blk_start(sid + t * NS)
                pltpu.make_async_copy(bufs[p],
                                      osum_hbm.at[cid, pl.ds(pb, CHUNK)],
                                      sems_x[p]).wait()
                pltpu.make_async_copy(cbufs[p],
                                      ocnt_hbm.at[cid, pl.ds(pb, CHUNK)],
                                      sems_c[p]).wait()

    return k(x, grp2d)


def _mlp_body(psum_ref, pcnt_ref, w1_ref, b1_ref, w2_ref, b2_ref, out_ref):
    s = psum_ref[0] + psum_ref[1]
    cnt = pcnt_ref[0, :, 0:1] + pcnt_ref[1, :, 0:1]
    means = s / jnp.maximum(cnt, 1.0)
    h = jnp.dot(means, w1_ref[...], preferred_element_type=jnp.float32)
    h = h + b1_ref[...][None, :]
    h = 0.5 * h * (1.0 + lax.erf(h * 0.7071067811865476))
    o = jnp.dot(h, w2_ref[...], preferred_element_type=jnp.float32)
    out_ref[...] = o + b2_ref[...][None, :]


def _tc_mlp(psum, pcnt, W1, b1, W2, b2):
    BG = 1000
    return pl.pallas_call(
        _mlp_body,
        grid=(G // BG,),
        in_specs=[
            pl.BlockSpec((NC, BG, D), lambda i: (0, i, 0)),
            pl.BlockSpec((NC, BG, CW), lambda i: (0, i, 0)),
            pl.BlockSpec((D, D), lambda i: (0, 0)),
            pl.BlockSpec((D,), lambda i: (0,)),
            pl.BlockSpec((D, C), lambda i: (0, 0)),
            pl.BlockSpec((C,), lambda i: (0,)),
        ],
        out_specs=pl.BlockSpec((BG, C), lambda i: (i, 0)),
        out_shape=jax.ShapeDtypeStruct((G, C), jnp.float32),
    )(psum, pcnt, W1, b1, W2, b2)


def kernel(x, group, W1, b1, W2, b2):
    grp2d = group.reshape(N // CHUNK, CHUNK)
    psum, pcnt = _sc_segment_sums(x, grp2d)
    return _tc_mlp(psum, pcnt, W1, b1, W2, b2)
